# baseline (speedup 1.0000x reference)
import functools

import jax
import jax.numpy as jnp
from jax.experimental import pallas as pl

NEIGH_HALF = 1.0
TP = 16
KPAD = 32


def _pool_kernel(ap_ref, ep_ref, oh_ref, h_ref, W_sp_ref, b_sp_ref,
                 W1_ref, b1_ref, W2_ref, b2_ref, out_ref, *, nc):
    e64 = W1_ref.shape[0] - h_ref.shape[1]
    W1_top = W1_ref[:e64, :]
    A = jnp.dot(W_sp_ref[...], W1_top, preferred_element_type=jnp.float32)
    base = (jnp.dot(h_ref[...], W1_ref[e64:, :],
                    preferred_element_type=jnp.float32)
            + jnp.dot(b_sp_ref[...], W1_top,
                      preferred_element_type=jnp.float32)
            + b1_ref[...])
    Ap = jnp.concatenate(
        [A, base, jnp.zeros((KPAD - 2 - base.shape[0], A.shape[1]),
                            jnp.float32)], axis=0).astype(jnp.bfloat16)

    d = ap_ref[...] - ep_ref[...]
    d = jnp.where(jnp.abs(d) > NEIGH_HALF, 0.0, d)
    Rp = (d + oh_ref[...]).astype(jnp.bfloat16)

    pre1 = jnp.dot(Rp, Ap, preferred_element_type=jnp.float32)
    x1 = jnp.maximum(pre1, 0).astype(jnp.bfloat16)
    y = jnp.dot(x1, W2_ref[...], preferred_element_type=jnp.float32)
    ymax = jnp.max(y.reshape(nc, -1, y.shape[1]), axis=0)
    out_ref[...] = jnp.maximum(ymax.astype(jnp.float32) + b2_ref[...], 0.0)


def kernel(h_states, end_pos, rel_pos, annotated_points, W_sp, b_sp, W1, b1,
           W2, b2, seq_start_end):
    del rel_pos, seq_start_end
    h = h_states.reshape(-1, h_states.shape[-1])
    B = h.shape[0]
    NC = annotated_points.shape[0]
    BN = W2.shape[1]
    NT = B // TP
    R = NC * TP

    ap_e = jnp.broadcast_to(annotated_points[None, :, None, :],
                            (NT, NC, TP, 2)).reshape(NT * R, 2)
    ep_e = jnp.broadcast_to(end_pos.reshape(NT, 1, TP, 2),
                            (NT, NC, TP, 2)).reshape(NT * R, 2)
    ap_e = jnp.pad(ap_e, ((0, 0), (0, KPAD - 2)))
    ep_e = jnp.pad(ep_e, ((0, 0), (0, KPAD - 2)))
    oh = (jax.lax.broadcasted_iota(jnp.int32, (R, KPAD), 1)
          == 2 + jnp.arange(R)[:, None] % TP).astype(jnp.float32)

    full = lambda shape: pl.BlockSpec(shape, lambda i: (0, 0))
    out = pl.pallas_call(
        functools.partial(_pool_kernel, nc=NC),
        grid=(NT,),
        in_specs=[
            pl.BlockSpec((R, KPAD), lambda i: (i, 0)),
            pl.BlockSpec((R, KPAD), lambda i: (i, 0)),
            full((R, KPAD)),
            pl.BlockSpec((TP, h.shape[1]), lambda i: (i, 0)),
            full(W_sp.shape),
            full((1, b_sp.shape[0])),
            full(W1.shape),
            full((1, b1.shape[0])),
            full(W2.shape),
            full((1, b2.shape[0])),
        ],
        out_specs=pl.BlockSpec((TP, BN), lambda i: (i, 0)),
        out_shape=jax.ShapeDtypeStruct((B, BN), jnp.float32),
    )(ap_e, ep_e, oh, h, W_sp, b_sp.reshape(1, -1), W1, b1.reshape(1, -1),
      W2.astype(jnp.bfloat16), b2.reshape(1, -1))
    return out


# speedup vs baseline: 1.5031x; 1.5031x over previous
import functools

import jax
import jax.numpy as jnp
from jax.experimental import pallas as pl

NEIGH_HALF = 1.0
TP = 16


def _pool_kernel(U_ref, ap_ref, ep_ref, h_ref, W_sp_ref, b_sp_ref,
                 W1_ref, b1_ref, W2_ref, b2_ref, out_ref, *, nc, kpad):
    tp = h_ref.shape[0]
    e64 = W1_ref.shape[0] - h_ref.shape[1]
    W1_top = W1_ref[:e64, :]
    A = jnp.dot(W_sp_ref[...], W1_top, preferred_element_type=jnp.float32)
    base = (jnp.dot(h_ref[...], W1_ref[e64:, :],
                    preferred_element_type=jnp.float32)
            + jnp.dot(b_sp_ref[...], W1_top,
                      preferred_element_type=jnp.float32)
            + b1_ref[...])
    Ap = jnp.concatenate(
        [A, base, jnp.zeros((kpad - 2 - tp, A.shape[1]), jnp.float32)],
        axis=0).astype(jnp.bfloat16)

    Vc = jnp.pad(ap_ref[...], ((0, 0), (0, kpad - 2)))
    eye = (jax.lax.broadcasted_iota(jnp.int32, (tp, kpad), 1)
           == 2 + jax.lax.broadcasted_iota(jnp.int32, (tp, kpad), 0)
           ).astype(jnp.float32)
    Vp = eye - jnp.pad(ep_ref[...], ((0, 0), (0, kpad - 2)))
    V = jnp.concatenate([Vc, Vp], axis=0)

    Rp = jnp.dot(U_ref[...], V, preferred_element_type=jnp.float32)
    Rp = jnp.where(jnp.abs(Rp) > NEIGH_HALF, 0.0, Rp).astype(jnp.bfloat16)

    pre1 = jnp.dot(Rp, Ap, preferred_element_type=jnp.float32)
    x1 = jnp.maximum(pre1, 0).astype(jnp.bfloat16)
    y = jnp.dot(x1, W2_ref[...], preferred_element_type=jnp.float32)
    ymax = jnp.max(y.reshape(nc, tp, y.shape[1]), axis=0)
    out_ref[...] = jnp.maximum(ymax + b2_ref[...], 0.0)


def kernel(h_states, end_pos, rel_pos, annotated_points, W_sp, b_sp, W1, b1,
           W2, b2, seq_start_end):
    del rel_pos, seq_start_end
    h = h_states.reshape(-1, h_states.shape[-1])
    B = h.shape[0]
    NC = annotated_points.shape[0]
    BN = W2.shape[1]
    NT = B // TP
    R = NC * TP
    KPAD = -(-(2 + TP) // 8) * 8

    ji = jnp.arange(R)[:, None]
    ci = jax.lax.broadcasted_iota(jnp.int32, (R, NC + TP), 1)
    U = ((ci == ji // TP) | (ci == NC + ji % TP)).astype(jnp.float32)

    full = lambda shape: pl.BlockSpec(shape, lambda i: (0, 0))
    out = pl.pallas_call(
        functools.partial(_pool_kernel, nc=NC, kpad=KPAD),
        grid=(NT,),
        in_specs=[
            full((R, NC + TP)),
            full((NC, 2)),
            pl.BlockSpec((TP, 2), lambda i: (i, 0)),
            pl.BlockSpec((TP, h.shape[1]), lambda i: (i, 0)),
            full(W_sp.shape),
            full((1, b_sp.shape[0])),
            full(W1.shape),
            full((1, b1.shape[0])),
            full(W2.shape),
            full((1, b2.shape[0])),
        ],
        out_specs=pl.BlockSpec((TP, BN), lambda i: (i, 0)),
        out_shape=jax.ShapeDtypeStruct((B, BN), jnp.float32),
    )(U, annotated_points, end_pos, h, W_sp, b_sp.reshape(1, -1), W1,
      b1.reshape(1, -1), W2.astype(jnp.bfloat16), b2.reshape(1, -1))
    return out
